# baseline (speedup 1.0000x reference)
import functools

import jax
import jax.numpy as jnp
from jax import lax
from jax.experimental import pallas as pl
from jax.experimental.pallas import tpu as pltpu
from jax.experimental.pallas import tpu_sc as plsc

_M, _N, _DIM = 64, 128, 256
_NUM = _M * _N
_ALPHA = 0.3
_SIGMA = max(_M, _N) / 2.0

_NC, _NS, _L = 2, 16, 16
_NW = _NC * _NS
_RPW = _NUM // _NW
_DRT = 2 * _RPW
_KD = _DIM // _L
_NB = 4
_RPB = _RPW // _NB
_PAD = _L + 1
_BIG = 2**30

_mesh = plsc.VectorSubcoreMesh(core_axis_name="c", subcore_axis_name="s")
_cparams = pltpu.CompilerParams(needs_layout_passes=False)


def _iota16():
    return lax.broadcasted_iota(jnp.int32, (_L,), 0)


def _tree_sum(vals):
    vals = list(vals)
    while len(vals) > 1:
        vals = [a + b for a, b in zip(vals[::2], vals[1::2])]
    return vals[0]


@functools.partial(
    pl.kernel,
    mesh=_mesh,
    out_type=(
        jax.ShapeDtypeStruct((_L,), jnp.int32),
        jax.ShapeDtypeStruct((_NUM, _DIM), jnp.float32),
        jax.ShapeDtypeStruct((_NS, 2 * _L), jnp.float32),
    ),
    scratch_types=[
        pltpu.VMEM((_DIM,), jnp.float32),
        pltpu.VMEM((_RPW, _DIM), jnp.float32),
        pltpu.VMEM((2, _RPB, _DIM), jnp.float32),
        pltpu.VMEM((_DRT * _PAD,), jnp.float32),
        pltpu.VMEM((2 * _L,), jnp.float32),
        pltpu.VMEM((_NS, 2 * _L), jnp.float32),
        pltpu.VMEM((_L,), jnp.int32),
        pltpu.SemaphoreType.DMA,
        pltpu.SemaphoreType.DMA,
        pltpu.SemaphoreType.DMA,
        pltpu.SemaphoreType.DMA,
        pltpu.SemaphoreType.DMA,
        pltpu.SemaphoreType.DMA,
        pltpu.SemaphoreType.DMA,
        pltpu.SemaphoreType.DMA,
    ],
    compiler_params=_cparams,
)
def _som_kernel(x_hbm, w_hbm, misc_hbm, out_hbm, exch_hbm,
                xv, wv, fv, dacc, pv, av, misc_v,
                sem_x, so0, so1, so2, so3, sf0, sf1, sem_o):
    c = lax.axis_index("c")
    s = lax.axis_index("s")
    wid = s * _NC + c
    base = wid * _RPW
    mbase = (s * _NC + (1 - c)) * _RPW
    own_sems = [so0, so1, so2, so3]
    mir_sems = [sf0, sf1]

    hx = pltpu.async_copy(x_hbm, xv, sem_x)
    hown = [
        pltpu.async_copy(
            w_hbm.at[pl.ds(base + b * _RPB, _RPB)],
            wv.at[pl.ds(b * _RPB, _RPB)],
            own_sems[b],
        )
        for b in range(_NB)
    ]
    hmir = {}
    for b in range(2):
        hmir[b] = pltpu.async_copy(
            w_hbm.at[pl.ds(mbase + b * _RPB, _RPB)],
            fv.at[b], mir_sems[b],
        )
    hx.wait()
    xs = [xv[pl.ds(k * _L, _L)] for k in range(_KD)]

    own_p0 = c * _RPW
    mir_p0 = (1 - c) * _RPW

    def dist_rows(src, r0, p0, rows):
        def half_body(hg, _):
            rr = r0 + hg * 8
            pp = p0 + hg * 8
            for l in range(8):
                r = rr + l
                es = [xs[k] - src[r, pl.ds(k * _L, _L)] for k in range(_KD)]
                acc = _tree_sum([e * e for e in es])
                dacc[pl.ds((pp + l) * _PAD, _L)] = acc
            return 0

        lax.fori_loop(0, rows // 8, half_body, 0)

    for b in range(_NB):
        hown[b].wait()
        dist_rows(wv, b * _RPB, own_p0 + b * _RPB, _RPB)
    for b in range(_NB):
        h = b % 2
        hmir[b].wait()
        dist_rows(fv.at[h], 0, mir_p0 + b * _RPB, _RPB)
        if b + 2 < _NB:
            hmir[b + 2] = pltpu.async_copy(
                w_hbm.at[pl.ds(mbase + (b + 2) * _RPB, _RPB)],
                fv.at[h], mir_sems[h],
            )

    lanes = _iota16()
    gbase = s * _DRT
    best_d = jnp.full((_L,), jnp.inf, jnp.float32)
    best_i = jnp.zeros((_L,), jnp.int32)
    for g in range(_DRT // _L):
        pos = jnp.int32(g * _L) + lanes
        flat = pos * _PAD
        d = _tree_sum([plsc.load_gather(dacc, [flat + k]) for k in range(_L)])
        take = d < best_d
        best_d = jnp.where(take, d, best_d)
        best_i = jnp.where(take, gbase + pos, best_i)

    pv[pl.ds(0, _L)] = best_d
    pv[pl.ds(_L, _L)] = plsc.bitcast(best_i, jnp.float32)
    pltpu.sync_copy(pv, exch_hbm.at[s])
    plsc.subcore_barrier()
    pltpu.sync_copy(exch_hbm, av)

    best_d = av[0, pl.ds(0, _L)]
    best_i = plsc.bitcast(av[0, pl.ds(_L, _L)], jnp.int32)
    for t in range(1, _NS):
        d = av[t, pl.ds(0, _L)]
        i = plsc.bitcast(av[t, pl.ds(_L, _L)], jnp.int32)
        take = d < best_d
        best_d = jnp.where(take, d, best_d)
        best_i = jnp.where(take, i, best_i)
    dmin = jnp.min(best_d)
    bmu = jnp.min(jnp.where(best_d == dmin, best_i, jnp.int32(_BIG)))
    bi = bmu // _N
    bj = bmu - bi * _N
    bif = bi.astype(jnp.float32)
    bjf = bj.astype(jnp.float32)

    ho = []
    for b in range(_NB):

        def upd_body(hg, _, b=b):
            rloc = b * _RPB + hg * 8
            rows = (base + rloc) + lanes
            rf = rows.astype(jnp.float32)
            rate = 1.0 - rf * jnp.float32(1.0 / _NUM)
            alpha_t = rate * jnp.float32(_ALPHA)
            sigma_t = rate * jnp.float32(_SIGMA)
            ri = rows // _N
            rj = rows - ri * _N
            di = bif - ri.astype(jnp.float32)
            dj = bjf - rj.astype(jnp.float32)
            ld2 = di * di + dj * dj
            h = jnp.exp(-ld2 / (2.0 * sigma_t * sigma_t))
            cvec = h * alpha_t
            for l in range(8):
                cc = cvec[l]
                r = rloc + l
                ws = [wv[r, pl.ds(k * _L, _L)] for k in range(_KD)]
                ys = [w + cc * (x - w) for w, x in zip(ws, xs)]
                for k in range(_KD):
                    wv[r, pl.ds(k * _L, _L)] = ys[k]
            return 0

        lax.fori_loop(0, _RPB // 8, upd_body, 0)
        ho.append(
            pltpu.async_copy(
                wv.at[pl.ds(b * _RPB, _RPB)],
                out_hbm.at[pl.ds(base + b * _RPB, _RPB)],
                sem_o,
            )
        )

    @pl.when(wid == 0)
    def _():
        lanes2 = _iota16()
        zero = jnp.zeros((_L,), jnp.int32)
        vec = jnp.where(lanes2 == 0, bmu, zero)
        vec = jnp.where(lanes2 == 1, bi, vec)
        vec = jnp.where(lanes2 == 2, bj, vec)
        misc_v[pl.ds(0, _L)] = vec
        pltpu.sync_copy(misc_v, misc_hbm)

    for h in ho:
        h.wait()


def kernel(x, weights, locations):
    del locations
    misc, new_w, _ = _som_kernel(x, weights)
    return misc[0], misc[1:3], new_w


# speedup vs baseline: 1.0835x; 1.0835x over previous
import functools

import jax
import jax.numpy as jnp
from jax import lax
from jax.experimental import pallas as pl
from jax.experimental.pallas import tpu as pltpu
from jax.experimental.pallas import tpu_sc as plsc

_M, _N, _DIM = 64, 128, 256
_NUM = _M * _N
_ALPHA = 0.3
_SIGMA = max(_M, _N) / 2.0

_NC, _NS, _L = 2, 16, 16
_NW = _NC * _NS
_RPW = _NUM // _NW
_DRT = 2 * _RPW
_KD = _DIM // _L
_NB = 4
_RPB = _RPW // _NB
_PAD = _L + 1
_BIG = 2**30

_mesh = plsc.VectorSubcoreMesh(core_axis_name="c", subcore_axis_name="s")
_cparams = pltpu.CompilerParams(needs_layout_passes=False)


def _iota16():
    return lax.broadcasted_iota(jnp.int32, (_L,), 0)


def _tree_sum(vals):
    vals = list(vals)
    while len(vals) > 1:
        vals = [a + b for a, b in zip(vals[::2], vals[1::2])]
    return vals[0]


@functools.partial(
    pl.kernel,
    mesh=_mesh,
    out_type=(
        jax.ShapeDtypeStruct((_L,), jnp.int32),
        jax.ShapeDtypeStruct((_NUM, _DIM), jnp.float32),
        jax.ShapeDtypeStruct((_NS, 2 * _L), jnp.float32),
    ),
    scratch_types=[
        pltpu.VMEM((_DIM,), jnp.float32),
        pltpu.VMEM((_RPW, _DIM), jnp.float32),
        pltpu.VMEM((2, _RPB, _DIM), jnp.float32),
        pltpu.VMEM((_DRT * _PAD,), jnp.float32),
        pltpu.VMEM((2 * _L,), jnp.float32),
        pltpu.VMEM((_NS, 2 * _L), jnp.float32),
        pltpu.VMEM((_L,), jnp.int32),
        pltpu.SemaphoreType.DMA,
        pltpu.SemaphoreType.DMA,
        pltpu.SemaphoreType.DMA,
        pltpu.SemaphoreType.DMA,
        pltpu.SemaphoreType.DMA,
        pltpu.SemaphoreType.DMA,
        pltpu.SemaphoreType.DMA,
        pltpu.SemaphoreType.DMA,
    ],
    compiler_params=_cparams,
)
def _som_kernel(x_hbm, w_hbm, misc_hbm, out_hbm, exch_hbm,
                xv, wv, fv, dacc, pv, av, misc_v,
                sem_x, so0, so1, so2, so3, sf0, sf1, sem_o):
    c = lax.axis_index("c")
    s = lax.axis_index("s")
    wid = s * _NC + c
    base = wid * _RPW
    mbase = (s * _NC + (1 - c)) * _RPW
    own_sems = [so0, so1, so2, so3]
    mir_sems = [sf0, sf1]

    hx = pltpu.async_copy(x_hbm, xv, sem_x)
    hown = [
        pltpu.async_copy(
            w_hbm.at[pl.ds(base + b * _RPB, _RPB)],
            wv.at[pl.ds(b * _RPB, _RPB)],
            own_sems[b],
        )
        for b in range(_NB)
    ]
    hmir = {}
    for b in range(2):
        hmir[b] = pltpu.async_copy(
            w_hbm.at[pl.ds(mbase + b * _RPB, _RPB)],
            fv.at[b], mir_sems[b],
        )
    hx.wait()
    xs = [xv[pl.ds(k * _L, _L)] for k in range(_KD)]

    own_p0 = c * _RPW
    mir_p0 = (1 - c) * _RPW

    def dist_rows(src, r0, p0, rows):
        def half_body(hg, _):
            rr = r0 + hg * 8
            pp = p0 + hg * 8
            accs = [jnp.zeros((_L,), jnp.float32) for _ in range(8)]
            for cb in range(_KD // 4):
                x4 = [xv[pl.ds((cb * 4 + j) * _L, _L)] for j in range(4)]
                for l in range(8):
                    w4 = [src[rr + l, pl.ds((cb * 4 + j) * _L, _L)]
                          for j in range(4)]
                    es = [x - w for x, w in zip(x4, w4)]
                    accs[l] = accs[l] + _tree_sum([e * e for e in es])
            for l in range(8):
                dacc[pl.ds((pp + l) * _PAD, _L)] = accs[l]
            return 0

        lax.fori_loop(0, rows // 8, half_body, 0)

    for b in range(_NB):
        hown[b].wait()
        dist_rows(wv, b * _RPB, own_p0 + b * _RPB, _RPB)
    for b in range(_NB):
        h = b % 2
        hmir[b].wait()
        dist_rows(fv.at[h], 0, mir_p0 + b * _RPB, _RPB)
        if b + 2 < _NB:
            hmir[b + 2] = pltpu.async_copy(
                w_hbm.at[pl.ds(mbase + (b + 2) * _RPB, _RPB)],
                fv.at[h], mir_sems[h],
            )

    lanes = _iota16()
    gbase = s * _DRT
    best_d = jnp.full((_L,), jnp.inf, jnp.float32)
    best_i = jnp.zeros((_L,), jnp.int32)
    for g in range(_DRT // _L):
        pos = jnp.int32(g * _L) + lanes
        flat = pos * _PAD
        d = _tree_sum([plsc.load_gather(dacc, [flat + k]) for k in range(_L)])
        take = d < best_d
        best_d = jnp.where(take, d, best_d)
        best_i = jnp.where(take, gbase + pos, best_i)

    pv[pl.ds(0, _L)] = best_d
    pv[pl.ds(_L, _L)] = plsc.bitcast(best_i, jnp.float32)
    pltpu.sync_copy(pv, exch_hbm.at[s])
    plsc.subcore_barrier()
    pltpu.sync_copy(exch_hbm, av)

    best_d = av[0, pl.ds(0, _L)]
    best_i = plsc.bitcast(av[0, pl.ds(_L, _L)], jnp.int32)
    for t in range(1, _NS):
        d = av[t, pl.ds(0, _L)]
        i = plsc.bitcast(av[t, pl.ds(_L, _L)], jnp.int32)
        take = d < best_d
        best_d = jnp.where(take, d, best_d)
        best_i = jnp.where(take, i, best_i)
    dmin = jnp.min(best_d)
    bmu = jnp.min(jnp.where(best_d == dmin, best_i, jnp.int32(_BIG)))
    bi = bmu // _N
    bj = bmu - bi * _N
    bif = bi.astype(jnp.float32)
    bjf = bj.astype(jnp.float32)

    ho = []
    for b in range(_NB):

        def upd_body(hg, _, b=b):
            rloc = b * _RPB + hg * 8
            rows = (base + rloc) + lanes
            rf = rows.astype(jnp.float32)
            rate = 1.0 - rf * jnp.float32(1.0 / _NUM)
            alpha_t = rate * jnp.float32(_ALPHA)
            sigma_t = rate * jnp.float32(_SIGMA)
            ri = rows // _N
            rj = rows - ri * _N
            di = bif - ri.astype(jnp.float32)
            dj = bjf - rj.astype(jnp.float32)
            ld2 = di * di + dj * dj
            h = jnp.exp(-ld2 / (2.0 * sigma_t * sigma_t))
            cvec = h * alpha_t
            cs = [cvec[l] for l in range(8)]
            for cb in range(_KD // 4):
                x4 = [xv[pl.ds((cb * 4 + j) * _L, _L)] for j in range(4)]
                for l in range(8):
                    r = rloc + l
                    for j in range(4):
                        sl = pl.ds((cb * 4 + j) * _L, _L)
                        w = wv[r, sl]
                        wv[r, sl] = w + cs[l] * (x4[j] - w)
            return 0

        lax.fori_loop(0, _RPB // 8, upd_body, 0)
        ho.append(
            pltpu.async_copy(
                wv.at[pl.ds(b * _RPB, _RPB)],
                out_hbm.at[pl.ds(base + b * _RPB, _RPB)],
                sem_o,
            )
        )

    @pl.when(wid == 0)
    def _():
        lanes2 = _iota16()
        zero = jnp.zeros((_L,), jnp.int32)
        vec = jnp.where(lanes2 == 0, bmu, zero)
        vec = jnp.where(lanes2 == 1, bi, vec)
        vec = jnp.where(lanes2 == 2, bj, vec)
        misc_v[pl.ds(0, _L)] = vec
        pltpu.sync_copy(misc_v, misc_hbm)

    for h in ho:
        h.wait()


def kernel(x, weights, locations):
    del locations
    misc, new_w, _ = _som_kernel(x, weights)
    return misc[0], misc[1:3], new_w
